# baseline (speedup 1.0000x reference)
import functools

import jax
import jax.numpy as jnp
from jax import lax
from jax.experimental import pallas as pl
from jax.experimental.pallas import tpu as pltpu
from jax.experimental.pallas import tpu_sc as plsc

N = 10000
D = 128
DH = 64
NC = 2
NS = 16
ROWS_PER_TILE = 628
N_PAD = NS * ROWS_PER_TILE
E = 320000
K = 256
CHUNKS = 80
E_PAD = NS * CHUNKS * K
CW = 16


def _sc_aggregate_body(with_counts, xlo_hbm, xhi_hbm, edges_hbm, zf_hbm,
                       zc_hbm, agglo_hbm, agghi_hbm, *refs):
    (cnt_hbm, src_v, dst_v, rows_v, rows2_v, ones_v, acc_sh, cnt_sh,
     sem, sem2) = refs
    c = lax.axis_index("c")
    s = lax.axis_index("s")

    base = s * ROWS_PER_TILE
    pltpu.sync_copy(zf_hbm.at[pl.ds(base, ROWS_PER_TILE)],
                    acc_sh.at[0, pl.ds(base, ROWS_PER_TILE)])

    def _orow(i, _):
        ones_v[0, i, pl.ds(0, 16)] = jnp.ones((16,), jnp.float32)
        return 0
    lax.fori_loop(0, K, _orow, 0)

    @pl.when(c == 0)
    def _():
        pltpu.sync_copy(zc_hbm.at[pl.ds(base, ROWS_PER_TILE)],
                        cnt_sh.at[0, pl.ds(base, ROWS_PER_TILE)])

    plsc.subcore_barrier()

    pltpu.sync_copy(edges_hbm.at[s], src_v)

    def _unpack(i, _):
        for k in range(K // 16):
            v = src_v[i, 0, pl.ds(k * 16, 16)]
            dst_v[i, 0, pl.ds(k * 16, 16)] = lax.bitwise_and(v, 16383)
            src_v[i, 0, pl.ds(k * 16, 16)] = lax.shift_right_logical(v, 14)
        return 0
    lax.fori_loop(0, CHUNKS, _unpack, 0)

    def _pair_c0(p, _):
        j = 2 * p
        g0 = pltpu.async_copy(xlo_hbm.at[src_v.at[j]], rows_v, sem)
        g1 = pltpu.async_copy(xlo_hbm.at[src_v.at[j + 1]], rows2_v, sem2)
        g0.wait()
        pltpu.sync_copy(rows_v, acc_sh.at[dst_v.at[j]], add=True)
        pltpu.sync_copy(ones_v, cnt_sh.at[dst_v.at[j]], add=True)
        g1.wait()
        pltpu.sync_copy(rows2_v, acc_sh.at[dst_v.at[j + 1]], add=True)
        pltpu.sync_copy(ones_v, cnt_sh.at[dst_v.at[j + 1]], add=True)
        return 0

    def _pair_c1(p, _):
        j = 2 * p
        g0 = pltpu.async_copy(xhi_hbm.at[src_v.at[j]], rows_v, sem)
        g1 = pltpu.async_copy(xhi_hbm.at[src_v.at[j + 1]], rows2_v, sem2)
        g0.wait()
        pltpu.sync_copy(rows_v, acc_sh.at[dst_v.at[j]], add=True)
        g1.wait()
        pltpu.sync_copy(rows2_v, acc_sh.at[dst_v.at[j + 1]], add=True)
        return 0

    @pl.when(c == 0)
    def _():
        lax.fori_loop(0, CHUNKS // 2, _pair_c0, 0)

    @pl.when(c == 1)
    def _():
        lax.fori_loop(0, CHUNKS // 2, _pair_c1, 0)

    plsc.subcore_barrier()

    @pl.when(c == 0)
    def _():
        pltpu.sync_copy(acc_sh.at[0, pl.ds(base, ROWS_PER_TILE)],
                        agglo_hbm.at[pl.ds(base, ROWS_PER_TILE)])
        if with_counts:
            pltpu.sync_copy(cnt_sh.at[0, pl.ds(base, ROWS_PER_TILE)],
                            cnt_hbm.at[pl.ds(base, ROWS_PER_TILE)])

    @pl.when(c == 1)
    def _():
        pltpu.sync_copy(acc_sh.at[0, pl.ds(base, ROWS_PER_TILE)],
                        agghi_hbm.at[pl.ds(base, ROWS_PER_TILE)])


def _make_sc_aggregate(with_counts):
    mesh = plsc.VectorSubcoreMesh(core_axis_name="c", subcore_axis_name="s")
    out_type = [
        jax.ShapeDtypeStruct((N_PAD, DH), jnp.float32),
        jax.ShapeDtypeStruct((N_PAD, DH), jnp.float32),
    ]
    out_type.append(jax.ShapeDtypeStruct((N_PAD, CW), jnp.float32))
    scratch = [
        pltpu.VMEM((CHUNKS, 1, K), jnp.int32),
        pltpu.VMEM((CHUNKS, 1, K), jnp.int32),
        pltpu.VMEM((1, K, DH), jnp.float32),
        pltpu.VMEM((1, K, DH), jnp.float32),
        pltpu.VMEM((1, K, CW), jnp.float32),
        pltpu.VMEM_SHARED((1, N_PAD, DH), jnp.float32),
        pltpu.VMEM_SHARED((1, N_PAD, CW), jnp.float32),
        pltpu.SemaphoreType.DMA,
        pltpu.SemaphoreType.DMA,
    ]
    out_type = tuple(out_type)
    return pl.kernel(
        functools.partial(_sc_aggregate_body, with_counts),
        out_type=out_type, mesh=mesh, scratch_types=scratch,
        compiler_params=pltpu.CompilerParams(use_tc_tiling_on_sc=False),
        name=f"sc_sage_aggregate_cnt{int(with_counts)}",
    )


_sc_agg_cnt = _make_sc_aggregate(True)

BR = 1000


def _tc_layer_body(act, agglo_ref, agghi_ref, cnt_ref, x_ref, wl_ref, bl_ref,
                   wr_ref, out_ref, *maybe_sig):
    agg = jnp.concatenate([agglo_ref[...], agghi_ref[...]], axis=1)
    cnt = cnt_ref[:, 0:1]
    mean = agg * (1.0 / jnp.clip(cnt, 1.0, None))
    out = (jnp.dot(mean, wl_ref[...], preferred_element_type=jnp.float32)
           + bl_ref[...]
           + jnp.dot(x_ref[...], wr_ref[...], preferred_element_type=jnp.float32))
    if act == "relu":
        out_ref[...] = jnp.maximum(out, 0.0)
    else:
        out_ref[...] = out
        maybe_sig[0][...] = jax.nn.sigmoid(out)


def _make_tc_layer(act):
    grid = (N // BR,)
    in_specs = [
        pl.BlockSpec((BR, DH), lambda i: (i, 0)),
        pl.BlockSpec((BR, DH), lambda i: (i, 0)),
        pl.BlockSpec((BR, CW), lambda i: (i, 0)),
        pl.BlockSpec((BR, D), lambda i: (i, 0)),
        pl.BlockSpec((D, D), lambda i: (0, 0)),
        pl.BlockSpec((1, D), lambda i: (0, 0)),
        pl.BlockSpec((D, D), lambda i: (0, 0)),
    ]
    nouts = 1 if act == "relu" else 2
    out_specs = tuple(pl.BlockSpec((BR, D), lambda i: (i, 0)) for _ in range(nouts))
    out_shape = tuple(jax.ShapeDtypeStruct((N, D), jnp.float32) for _ in range(nouts))
    return pl.pallas_call(
        functools.partial(_tc_layer_body, act),
        grid=grid, in_specs=in_specs, out_specs=out_specs,
        out_shape=out_shape,
    )


_tc_layer_relu = _make_tc_layer("relu")
_tc_layer_sig = _make_tc_layer("sig")


def kernel(x, edge_index, Wl0, bl0, Wr0, Wl1, bl1, Wr1):
    src = edge_index[0]
    dst = edge_index[1]
    pad = E_PAD - E
    packed = src * 16384 + dst
    edges = jnp.concatenate(
        [packed, jnp.full((pad,), N, jnp.int32)]).reshape(NS, CHUNKS, 1, K)

    xlo, xhi = x[:, :DH], x[:, DH:]
    zf = jnp.zeros((N_PAD, DH), jnp.float32)
    zc = jnp.zeros((N_PAD, CW), jnp.float32)
    agg0lo, agg0hi, cnt = _sc_agg_cnt(xlo[None], xhi[None], edges, zf, zc)
    (h,) = _tc_layer_relu(agg0lo, agg0hi, cnt, x, Wl0, bl0.reshape(1, D), Wr0)
    agg1lo, agg1hi, _ = _sc_agg_cnt(h[:, :DH][None], h[:, DH:][None], edges,
                                    zf, zc)
    out, sig = _tc_layer_sig(agg1lo, agg1hi, cnt, h, Wl1, bl1.reshape(1, D), Wr1)
    return (out, sig)


# speedup vs baseline: 1.0325x; 1.0325x over previous
import functools

import jax
import jax.numpy as jnp
from jax import lax
from jax.experimental import pallas as pl
from jax.experimental.pallas import tpu as pltpu
from jax.experimental.pallas import tpu_sc as plsc

N = 10000
D = 128
DH = 64
NC = 2
NS = 16
ROWS_PER_TILE = 628
N_PAD = NS * ROWS_PER_TILE
E = 320000
K = 512
CHUNKS = 40
E_PAD = NS * CHUNKS * K
CW = 8


def _sc_aggregate_body(with_counts, xlo_hbm, xhi_hbm, edges_hbm, zf_hbm,
                       zc_hbm, ones_hbm, agglo_hbm, agghi_hbm, *refs):
    (cnt_hbm, src_v, dst_v, rows_v, ones_v, acc_sh, cnt_sh, sem) = refs
    c = lax.axis_index("c")
    s = lax.axis_index("s")

    base = s * ROWS_PER_TILE
    pltpu.sync_copy(zf_hbm.at[pl.ds(base, ROWS_PER_TILE)],
                    acc_sh.at[0, pl.ds(base, ROWS_PER_TILE)])

    pltpu.sync_copy(ones_hbm, ones_v)

    @pl.when(c == 0)
    def _():
        pltpu.sync_copy(zc_hbm.at[pl.ds(base, ROWS_PER_TILE)],
                        cnt_sh.at[0, pl.ds(base, ROWS_PER_TILE)])

    plsc.subcore_barrier()

    pltpu.sync_copy(edges_hbm.at[s], src_v)

    def _unpack(i, _):
        for k in range(K // 16):
            v = src_v[i, 0, pl.ds(k * 16, 16)]
            dst_v[i, 0, pl.ds(k * 16, 16)] = lax.bitwise_and(v, 16383)
            src_v[i, 0, pl.ds(k * 16, 16)] = lax.shift_right_logical(v, 14)
        return 0
    lax.fori_loop(0, CHUNKS, _unpack, 0)

    def _chunk_c0(j, _):
        pltpu.async_copy(xlo_hbm.at[src_v.at[j]], rows_v, sem).wait()
        pltpu.sync_copy(rows_v, acc_sh.at[dst_v.at[j]], add=True)
        pltpu.sync_copy(ones_v, cnt_sh.at[dst_v.at[j]], add=True)
        return 0

    def _chunk_c1(j, _):
        pltpu.async_copy(xhi_hbm.at[src_v.at[j]], rows_v, sem).wait()
        pltpu.sync_copy(rows_v, acc_sh.at[dst_v.at[j]], add=True)
        return 0

    @pl.when(c == 0)
    def _():
        lax.fori_loop(0, CHUNKS, _chunk_c0, 0)

    @pl.when(c == 1)
    def _():
        lax.fori_loop(0, CHUNKS, _chunk_c1, 0)

    plsc.subcore_barrier()

    @pl.when(c == 0)
    def _():
        pltpu.sync_copy(acc_sh.at[0, pl.ds(base, ROWS_PER_TILE)],
                        agglo_hbm.at[pl.ds(base, ROWS_PER_TILE)])
        if with_counts:
            pltpu.sync_copy(cnt_sh.at[0, pl.ds(base, ROWS_PER_TILE)],
                            cnt_hbm.at[pl.ds(base, ROWS_PER_TILE)])

    @pl.when(c == 1)
    def _():
        pltpu.sync_copy(acc_sh.at[0, pl.ds(base, ROWS_PER_TILE)],
                        agghi_hbm.at[pl.ds(base, ROWS_PER_TILE)])


def _make_sc_aggregate(with_counts):
    mesh = plsc.VectorSubcoreMesh(core_axis_name="c", subcore_axis_name="s")
    out_type = [
        jax.ShapeDtypeStruct((N_PAD, DH), jnp.float32),
        jax.ShapeDtypeStruct((N_PAD, DH), jnp.float32),
    ]
    out_type.append(jax.ShapeDtypeStruct((N_PAD, CW), jnp.float32))
    scratch = [
        pltpu.VMEM((CHUNKS, 1, K), jnp.int32),
        pltpu.VMEM((CHUNKS, 1, K), jnp.int32),
        pltpu.VMEM((1, K, DH), jnp.float32),
        pltpu.VMEM((1, K, CW), jnp.float32),
        pltpu.VMEM_SHARED((1, N_PAD, DH), jnp.float32),
        pltpu.VMEM_SHARED((1, N_PAD, CW), jnp.float32),
        pltpu.SemaphoreType.DMA,
    ]
    out_type = tuple(out_type)
    return pl.kernel(
        functools.partial(_sc_aggregate_body, with_counts),
        out_type=out_type, mesh=mesh, scratch_types=scratch,
        compiler_params=pltpu.CompilerParams(use_tc_tiling_on_sc=False),
        name=f"sc_sage_aggregate_cnt{int(with_counts)}",
    )


_sc_agg_cnt = _make_sc_aggregate(True)

BR = 1000


def _tc_layer_body(act, agglo_ref, agghi_ref, cnt_ref, x_ref, wl_ref, bl_ref,
                   wr_ref, out_ref, *maybe_sig):
    agg = jnp.concatenate([agglo_ref[...], agghi_ref[...]], axis=1)
    cnt = cnt_ref[:, 0:1]
    mean = agg * (1.0 / jnp.clip(cnt, 1.0, None))
    out = (jnp.dot(mean, wl_ref[...], preferred_element_type=jnp.float32)
           + bl_ref[...]
           + jnp.dot(x_ref[...], wr_ref[...], preferred_element_type=jnp.float32))
    if act == "relu":
        out_ref[...] = jnp.maximum(out, 0.0)
    else:
        out_ref[...] = out
        maybe_sig[0][...] = jax.nn.sigmoid(out)


def _make_tc_layer(act):
    grid = (N // BR,)
    in_specs = [
        pl.BlockSpec((BR, DH), lambda i: (i, 0)),
        pl.BlockSpec((BR, DH), lambda i: (i, 0)),
        pl.BlockSpec((BR, CW), lambda i: (i, 0)),
        pl.BlockSpec((BR, D), lambda i: (i, 0)),
        pl.BlockSpec((D, D), lambda i: (0, 0)),
        pl.BlockSpec((1, D), lambda i: (0, 0)),
        pl.BlockSpec((D, D), lambda i: (0, 0)),
    ]
    nouts = 1 if act == "relu" else 2
    out_specs = tuple(pl.BlockSpec((BR, D), lambda i: (i, 0)) for _ in range(nouts))
    out_shape = tuple(jax.ShapeDtypeStruct((N, D), jnp.float32) for _ in range(nouts))
    return pl.pallas_call(
        functools.partial(_tc_layer_body, act),
        grid=grid, in_specs=in_specs, out_specs=out_specs,
        out_shape=out_shape,
    )


_tc_layer_relu = _make_tc_layer("relu")
_tc_layer_sig = _make_tc_layer("sig")


def kernel(x, edge_index, Wl0, bl0, Wr0, Wl1, bl1, Wr1):
    src = edge_index[0]
    dst = edge_index[1]
    pad = E_PAD - E
    packed = src * 16384 + dst
    edges = jnp.concatenate(
        [packed, jnp.full((pad,), N, jnp.int32)]).reshape(NS, CHUNKS, 1, K)

    xlo, xhi = x[:, :DH], x[:, DH:]
    zf = jnp.zeros((N_PAD, DH), jnp.float32)
    zc = jnp.zeros((N_PAD, CW), jnp.float32)
    on = jnp.ones((1, K, CW), jnp.float32)
    agg0lo, agg0hi, cnt = _sc_agg_cnt(xlo[None], xhi[None], edges, zf, zc, on)
    (h,) = _tc_layer_relu(agg0lo, agg0hi, cnt, x, Wl0, bl0.reshape(1, D), Wr0)
    agg1lo, agg1hi, _ = _sc_agg_cnt(h[:, :DH][None], h[:, DH:][None], edges,
                                    zf, zc, on)
    out, sig = _tc_layer_sig(agg1lo, agg1hi, cnt, h, Wl1, bl1.reshape(1, D), Wr1)
    return (out, sig)


# speedup vs baseline: 1.5138x; 1.4661x over previous
import functools

import jax
import jax.numpy as jnp
from jax import lax
from jax.experimental import pallas as pl
from jax.experimental.pallas import tpu as pltpu
from jax.experimental.pallas import tpu_sc as plsc

N = 10000
D = 128
DH = 64
NC = 2
NS = 16
ROWS_PER_TILE = 628
N_PAD = NS * ROWS_PER_TILE
E = 320000
K = 448
CHUNKS = 45
E_PAD = NS * CHUNKS * K
CW = 8


def _sc_aggregate_body(with_counts, xlo_hbm, xhi_hbm, edges_hbm, zf_hbm,
                       zc_hbm, ones_hbm, agglo_hbm, agghi_hbm, *refs):
    (cnt_hbm, src_v, dst_v, rows_v, ones_v, acc_sh, cnt_sh, sem) = refs
    c = lax.axis_index("c")
    s = lax.axis_index("s")

    base = s * ROWS_PER_TILE
    pltpu.sync_copy(zf_hbm.at[pl.ds(base, ROWS_PER_TILE)],
                    acc_sh.at[0, pl.ds(base, ROWS_PER_TILE)])

    pltpu.sync_copy(ones_hbm, ones_v)

    @pl.when(c == 0)
    def _():
        pltpu.sync_copy(zc_hbm.at[pl.ds(base, ROWS_PER_TILE)],
                        cnt_sh.at[0, pl.ds(base, ROWS_PER_TILE)])

    plsc.subcore_barrier()

    pltpu.sync_copy(edges_hbm.at[s], src_v)

    def _unpack(i, _):
        for k in range(K // 16):
            v = src_v[i, 0, pl.ds(k * 16, 16)]
            dst_v[i, 0, pl.ds(k * 16, 16)] = lax.bitwise_and(v, 16383)
            src_v[i, 0, pl.ds(k * 16, 16)] = lax.shift_right_logical(v, 14)
        return 0
    lax.fori_loop(0, CHUNKS, _unpack, 0)

    def _chunk_c0(j, _):
        pltpu.async_copy(xlo_hbm.at[src_v.at[j]], rows_v, sem).wait()
        pltpu.sync_copy(rows_v, acc_sh.at[dst_v.at[j]], add=True)
        pltpu.sync_copy(ones_v, cnt_sh.at[dst_v.at[j]], add=True)
        return 0

    def _chunk_c1(j, _):
        pltpu.async_copy(xhi_hbm.at[src_v.at[j]], rows_v, sem).wait()
        pltpu.sync_copy(rows_v, acc_sh.at[dst_v.at[j]], add=True)
        return 0

    @pl.when(c == 0)
    def _():
        lax.fori_loop(0, CHUNKS, _chunk_c0, 0)

    @pl.when(c == 1)
    def _():
        lax.fori_loop(0, CHUNKS, _chunk_c1, 0)

    plsc.subcore_barrier()

    @pl.when(c == 0)
    def _():
        pltpu.sync_copy(acc_sh.at[0, pl.ds(base, ROWS_PER_TILE)],
                        agglo_hbm.at[pl.ds(base, ROWS_PER_TILE)])
        if with_counts:
            pltpu.sync_copy(cnt_sh.at[0, pl.ds(base, ROWS_PER_TILE)],
                            cnt_hbm.at[pl.ds(base, ROWS_PER_TILE)])

    @pl.when(c == 1)
    def _():
        pltpu.sync_copy(acc_sh.at[0, pl.ds(base, ROWS_PER_TILE)],
                        agghi_hbm.at[pl.ds(base, ROWS_PER_TILE)])


def _make_sc_aggregate(with_counts):
    mesh = plsc.VectorSubcoreMesh(core_axis_name="c", subcore_axis_name="s")
    out_type = [
        jax.ShapeDtypeStruct((N_PAD, DH), jnp.float32),
        jax.ShapeDtypeStruct((N_PAD, DH), jnp.float32),
    ]
    out_type.append(jax.ShapeDtypeStruct((N_PAD, CW), jnp.float32))
    scratch = [
        pltpu.VMEM((CHUNKS, 1, K), jnp.int32),
        pltpu.VMEM((CHUNKS, 1, K), jnp.int32),
        pltpu.VMEM((1, K, DH), jnp.float32),
        pltpu.VMEM((1, K, CW), jnp.float32),
        pltpu.VMEM_SHARED((1, N_PAD, DH), jnp.float32),
        pltpu.VMEM_SHARED((1, N_PAD, CW), jnp.float32),
        pltpu.SemaphoreType.DMA,
    ]
    out_type = tuple(out_type)
    return pl.kernel(
        functools.partial(_sc_aggregate_body, with_counts),
        out_type=out_type, mesh=mesh, scratch_types=scratch,
        compiler_params=pltpu.CompilerParams(use_tc_tiling_on_sc=False),
        name=f"sc_sage_aggregate_cnt{int(with_counts)}",
    )


_sc_agg_cnt = _make_sc_aggregate(True)

BR = 1000


def _tc_layer_body(act, agglo_ref, agghi_ref, cnt_ref, x_ref, wl_ref, bl_ref,
                   wr_ref, out_ref, *maybe_sig):
    agg = jnp.concatenate([agglo_ref[...], agghi_ref[...]], axis=1)
    cnt = cnt_ref[:, 0:1]
    mean = agg * (1.0 / jnp.clip(cnt, 1.0, None))
    out = (jnp.dot(mean, wl_ref[...], preferred_element_type=jnp.float32)
           + bl_ref[...]
           + jnp.dot(x_ref[...], wr_ref[...], preferred_element_type=jnp.float32))
    if act == "relu":
        out_ref[...] = jnp.maximum(out, 0.0)
    else:
        out_ref[...] = out
        maybe_sig[0][...] = jax.nn.sigmoid(out)


def _make_tc_layer(act):
    grid = (N // BR,)
    in_specs = [
        pl.BlockSpec((BR, DH), lambda i: (i, 0)),
        pl.BlockSpec((BR, DH), lambda i: (i, 0)),
        pl.BlockSpec((BR, CW), lambda i: (i, 0)),
        pl.BlockSpec((BR, D), lambda i: (i, 0)),
        pl.BlockSpec((D, D), lambda i: (0, 0)),
        pl.BlockSpec((1, D), lambda i: (0, 0)),
        pl.BlockSpec((D, D), lambda i: (0, 0)),
    ]
    nouts = 1 if act == "relu" else 2
    out_specs = tuple(pl.BlockSpec((BR, D), lambda i: (i, 0)) for _ in range(nouts))
    out_shape = tuple(jax.ShapeDtypeStruct((N, D), jnp.float32) for _ in range(nouts))
    return pl.pallas_call(
        functools.partial(_tc_layer_body, act),
        grid=grid, in_specs=in_specs, out_specs=out_specs,
        out_shape=out_shape,
    )


_tc_layer_relu = _make_tc_layer("relu")
_tc_layer_sig = _make_tc_layer("sig")


def kernel(x, edge_index, Wl0, bl0, Wr0, Wl1, bl1, Wr1):
    src = edge_index[0]
    dst = edge_index[1]
    pad = E_PAD - E
    packed = src * 16384 + dst
    edges = jnp.concatenate(
        [packed, jnp.full((pad,), N, jnp.int32)]).reshape(NS, CHUNKS, 1, K)

    xlo, xhi = x[:, :DH], x[:, DH:]
    zf = jnp.zeros((N_PAD, DH), jnp.float32)
    zc = jnp.zeros((N_PAD, CW), jnp.float32)
    on = jnp.ones((1, K, CW), jnp.float32)
    agg0lo, agg0hi, cnt = _sc_agg_cnt(xlo[None], xhi[None], edges, zf, zc, on)
    (h,) = _tc_layer_relu(agg0lo, agg0hi, cnt, x, Wl0, bl0.reshape(1, D), Wr0)
    agg1lo, agg1hi, _ = _sc_agg_cnt(h[:, :DH][None], h[:, DH:][None], edges,
                                    zf, zc, on)
    out, sig = _tc_layer_sig(agg1lo, agg1hi, cnt, h, Wl1, bl1.reshape(1, D), Wr1)
    return (out, sig)


# speedup vs baseline: 1.5282x; 1.0095x over previous
import functools

import jax
import jax.numpy as jnp
from jax import lax
from jax.experimental import pallas as pl
from jax.experimental.pallas import tpu as pltpu
from jax.experimental.pallas import tpu_sc as plsc

N = 10000
D = 128
DH = 64
NC = 2
NS = 16
ROWS_PER_TILE = 628
N_PAD = NS * ROWS_PER_TILE
E = 320000
K = 480
CHUNKS = 42
E_PAD = NS * CHUNKS * K
CW = 8


def _sc_aggregate_body(with_counts, xlo_hbm, xhi_hbm, edges_hbm, zf_hbm,
                       zc_hbm, ones_hbm, agglo_hbm, agghi_hbm, *refs):
    (cnt_hbm, src_v, dst_v, rows_v, ones_v, acc_sh, cnt_sh, sem) = refs
    c = lax.axis_index("c")
    s = lax.axis_index("s")

    base = s * ROWS_PER_TILE
    pltpu.sync_copy(zf_hbm.at[pl.ds(base, ROWS_PER_TILE)],
                    acc_sh.at[0, pl.ds(base, ROWS_PER_TILE)])

    pltpu.sync_copy(ones_hbm, ones_v)

    @pl.when(c == 0)
    def _():
        pltpu.sync_copy(zc_hbm.at[pl.ds(base, ROWS_PER_TILE)],
                        cnt_sh.at[0, pl.ds(base, ROWS_PER_TILE)])

    plsc.subcore_barrier()

    pltpu.sync_copy(edges_hbm.at[s], src_v)

    def _unpack(i, _):
        for k in range(K // 16):
            v = src_v[i, 0, pl.ds(k * 16, 16)]
            dst_v[i, 0, pl.ds(k * 16, 16)] = lax.bitwise_and(v, 16383)
            src_v[i, 0, pl.ds(k * 16, 16)] = lax.shift_right_logical(v, 14)
        return 0
    lax.fori_loop(0, CHUNKS, _unpack, 0)

    def _chunk_c0(j, _):
        pltpu.async_copy(xlo_hbm.at[src_v.at[j]], rows_v, sem).wait()
        pltpu.sync_copy(rows_v, acc_sh.at[dst_v.at[j]], add=True)
        pltpu.sync_copy(ones_v, cnt_sh.at[dst_v.at[j]], add=True)
        return 0

    def _chunk_c1(j, _):
        pltpu.async_copy(xhi_hbm.at[src_v.at[j]], rows_v, sem).wait()
        pltpu.sync_copy(rows_v, acc_sh.at[dst_v.at[j]], add=True)
        return 0

    @pl.when(c == 0)
    def _():
        lax.fori_loop(0, CHUNKS, _chunk_c0, 0)

    @pl.when(c == 1)
    def _():
        lax.fori_loop(0, CHUNKS, _chunk_c1, 0)

    plsc.subcore_barrier()

    @pl.when(c == 0)
    def _():
        pltpu.sync_copy(acc_sh.at[0, pl.ds(base, ROWS_PER_TILE)],
                        agglo_hbm.at[pl.ds(base, ROWS_PER_TILE)])
        if with_counts:
            pltpu.sync_copy(cnt_sh.at[0, pl.ds(base, ROWS_PER_TILE)],
                            cnt_hbm.at[pl.ds(base, ROWS_PER_TILE)])

    @pl.when(c == 1)
    def _():
        pltpu.sync_copy(acc_sh.at[0, pl.ds(base, ROWS_PER_TILE)],
                        agghi_hbm.at[pl.ds(base, ROWS_PER_TILE)])


def _make_sc_aggregate(with_counts):
    mesh = plsc.VectorSubcoreMesh(core_axis_name="c", subcore_axis_name="s")
    out_type = [
        jax.ShapeDtypeStruct((N_PAD, DH), jnp.float32),
        jax.ShapeDtypeStruct((N_PAD, DH), jnp.float32),
    ]
    out_type.append(jax.ShapeDtypeStruct((N_PAD, CW), jnp.float32))
    scratch = [
        pltpu.VMEM((CHUNKS, 1, K), jnp.int32),
        pltpu.VMEM((CHUNKS, 1, K), jnp.int32),
        pltpu.VMEM((1, K, DH), jnp.float32),
        pltpu.VMEM((1, K, CW), jnp.float32),
        pltpu.VMEM_SHARED((1, N_PAD, DH), jnp.float32),
        pltpu.VMEM_SHARED((1, N_PAD, CW), jnp.float32),
        pltpu.SemaphoreType.DMA,
    ]
    out_type = tuple(out_type)
    return pl.kernel(
        functools.partial(_sc_aggregate_body, with_counts),
        out_type=out_type, mesh=mesh, scratch_types=scratch,
        compiler_params=pltpu.CompilerParams(use_tc_tiling_on_sc=False),
        name=f"sc_sage_aggregate_cnt{int(with_counts)}",
    )


_sc_agg_cnt = _make_sc_aggregate(True)

BR = 1000


def _tc_layer_body(act, agglo_ref, agghi_ref, cnt_ref, x_ref, wl_ref, bl_ref,
                   wr_ref, out_ref, *maybe_sig):
    agg = jnp.concatenate([agglo_ref[...], agghi_ref[...]], axis=1)
    cnt = cnt_ref[:, 0:1]
    mean = agg * (1.0 / jnp.clip(cnt, 1.0, None))
    out = (jnp.dot(mean, wl_ref[...], preferred_element_type=jnp.float32)
           + bl_ref[...]
           + jnp.dot(x_ref[...], wr_ref[...], preferred_element_type=jnp.float32))
    if act == "relu":
        out_ref[...] = jnp.maximum(out, 0.0)
    else:
        out_ref[...] = out
        maybe_sig[0][...] = jax.nn.sigmoid(out)


def _make_tc_layer(act):
    grid = (N // BR,)
    in_specs = [
        pl.BlockSpec((BR, DH), lambda i: (i, 0)),
        pl.BlockSpec((BR, DH), lambda i: (i, 0)),
        pl.BlockSpec((BR, CW), lambda i: (i, 0)),
        pl.BlockSpec((BR, D), lambda i: (i, 0)),
        pl.BlockSpec((D, D), lambda i: (0, 0)),
        pl.BlockSpec((1, D), lambda i: (0, 0)),
        pl.BlockSpec((D, D), lambda i: (0, 0)),
    ]
    nouts = 1 if act == "relu" else 2
    out_specs = tuple(pl.BlockSpec((BR, D), lambda i: (i, 0)) for _ in range(nouts))
    out_shape = tuple(jax.ShapeDtypeStruct((N, D), jnp.float32) for _ in range(nouts))
    return pl.pallas_call(
        functools.partial(_tc_layer_body, act),
        grid=grid, in_specs=in_specs, out_specs=out_specs,
        out_shape=out_shape,
    )


_tc_layer_relu = _make_tc_layer("relu")
_tc_layer_sig = _make_tc_layer("sig")


def kernel(x, edge_index, Wl0, bl0, Wr0, Wl1, bl1, Wr1):
    src = edge_index[0]
    dst = edge_index[1]
    pad = E_PAD - E
    packed = src * 16384 + dst
    edges = jnp.concatenate(
        [packed, jnp.full((pad,), N, jnp.int32)]).reshape(NS, CHUNKS, 1, K)

    xlo, xhi = x[:, :DH], x[:, DH:]
    zf = jnp.zeros((N_PAD, DH), jnp.float32)
    zc = jnp.zeros((N_PAD, CW), jnp.float32)
    on = jnp.ones((1, K, CW), jnp.float32)
    agg0lo, agg0hi, cnt = _sc_agg_cnt(xlo[None], xhi[None], edges, zf, zc, on)
    (h,) = _tc_layer_relu(agg0lo, agg0hi, cnt, x, Wl0, bl0.reshape(1, D), Wr0)
    agg1lo, agg1hi, _ = _sc_agg_cnt(h[:, :DH][None], h[:, DH:][None], edges,
                                    zf, zc, on)
    out, sig = _tc_layer_sig(agg1lo, agg1hi, cnt, h, Wl1, bl1.reshape(1, D), Wr1)
    return (out, sig)
